# baseline (speedup 1.0000x reference)
import functools

import jax
import jax.numpy as jnp
from jax import lax
from jax.experimental import pallas as pl
from jax.experimental.pallas import tpu as pltpu
from jax.experimental.pallas import tpu_sc as plsc


@functools.lru_cache(maxsize=None)
def _build_sc_kernel(B, H, W, C, P):
    info = plsc.get_sparse_core_info()
    NS, L = info.num_subcores, info.num_lanes
    NW = NS
    R = B * P
    rpw = R // NW
    assert R % NW == 0 and P % rpw == 0 and rpw % 128 == 0 and C % L == 0
    n_chunks = rpw // 128
    HW = H * W

    mesh = plsc.VectorSubcoreMesh(
        core_axis_name="c", subcore_axis_name="s", num_cores=1)

    @functools.partial(
        pl.kernel,
        mesh=mesh,
        out_type=jax.ShapeDtypeStruct((R, C), jnp.float32),
        compiler_params=pltpu.CompilerParams(
            needs_layout_passes=False, use_tc_tiling_on_sc=False
        ),
        scratch_types=[
            pltpu.VMEM((rpw,), jnp.int32),
            pltpu.VMEM((rpw,), jnp.int32),
            pltpu.VMEM((rpw,), jnp.float32),
            pltpu.VMEM((n_chunks, 128), jnp.int32),
            pltpu.VMEM((rpw, C), jnp.float32),
            pltpu.SemaphoreType.DMA,
        ],
    )
    def sc_kernel(table, yy, xx, mm, out, y_v, x_v, m_v, idx_v, rows_v, sem):
        wid = lax.axis_index("s")
        base = wid * rpw
        pltpu.sync_copy(yy.at[pl.ds(base, rpw)], y_v)
        pltpu.sync_copy(xx.at[pl.ds(base, rpw)], x_v)
        pltpu.sync_copy(mm.at[pl.ds(base, rpw)], m_v)

        b_off = (base // P) * HW
        per_row = 128 // L
        for k in range(rpw // L):
            yv = y_v[pl.ds(k * L, L)]
            xv = x_v[pl.ds(k * L, L)]
            idx_v[k // per_row, pl.ds((k % per_row) * L, L)] = (
                yv * W + xv + b_off
            )

        copies = [
            pltpu.async_copy(
                table.at[idx_v.at[j]],
                rows_v.at[pl.ds(j * 128, 128)],
                sem,
            )
            for j in range(n_chunks)
        ]
        for c in copies:
            c.wait()

        def mul_body(r, carry):
            m16 = plsc.load_gather(m_v, [lax.broadcast(r, (L,))])
            for d in range(C // L):
                rows_v[r, pl.ds(d * L, L)] = rows_v[r, pl.ds(d * L, L)] * m16
            return carry

        lax.fori_loop(0, rpw, mul_body, 0)

        pltpu.sync_copy(rows_v, out.at[pl.ds(base, rpw)])

    return sc_kernel


def kernel(embeddings, coords, mask):
    B, H, W, C = embeddings.shape
    P = coords.shape[1]
    Bh = B // 2
    c32 = coords.astype(jnp.int32)
    k = _build_sc_kernel(Bh, H, W, C, P)
    outs = []
    for h in range(2):
        emb_h = embeddings[h * Bh:(h + 1) * Bh].reshape(Bh * H * W, C)
        c_h = c32[h * Bh:(h + 1) * Bh]
        yy = c_h[..., 0].reshape(-1)
        xx = c_h[..., 1].reshape(-1)
        mm = mask[h * Bh:(h + 1) * Bh].reshape(-1)
        outs.append(k(emb_h, yy, xx, mm).reshape(Bh, P, C))
    return jnp.concatenate(outs, axis=0)
